# baseline (speedup 1.0000x reference)
import functools

import jax
import jax.numpy as jnp
import numpy as np
from jax.experimental import pallas as pl
from jax.experimental.pallas import tpu as pltpu

_BINS = 10
_ALPHA = 0.75
_EDGES = [np.float32(i) / np.float32(10.0) for i in range(_BINS + 1)]


def _ghm_body(x_ref, t_ref, out_ref, *, blk_rows):
    nchunks = blk_rows // 8

    def chunk(r0, accs):
        x = x_ref[pl.ds(r0, 8), :]
        t = t_ref[pl.ds(r0, 8), :]
        ax = jnp.abs(x)
        en = jnp.exp(-ax)
        loss = jnp.maximum(x, 0.0) - x * t + jnp.log1p(en)
        p1 = 1.0 / (1.0 + en)
        pred = jnp.where(x >= 0.0, p1, en * p1)
        g = jnp.abs(pred - t)

        def fold(v):
            return v[:, 0:128] + v[:, 128:256]

        new = list(accs)
        new[0] = new[0] + fold(loss)
        for i in range(1, _BINS + 1):
            m = g >= _EDGES[i]
            if i < _BINS:
                new[i] = new[i] + fold(jnp.where(m, loss, 0.0))
            new[9 + i] = new[9 + i] + fold(jnp.where(m, 1.0, 0.0))
        return tuple(new)

    zero = jnp.zeros((8, 128), jnp.float32)
    accs = (zero,) * 20
    for c in range(nchunks):
        accs = chunk(8 * c, accs)
    for j in range(20):
        out_ref[0, 8 * j:8 * j + 8, :] = accs[j]


def kernel(x, target):
    n = x.size
    cols = 256
    rows = n // cols
    blk_rows = min(2048, rows)
    grid = rows // blk_rows

    xr = x.reshape(rows, cols)
    tr = target.reshape(rows, cols)

    out = pl.pallas_call(
        functools.partial(_ghm_body, blk_rows=blk_rows),
        grid=(grid,),
        in_specs=[
            pl.BlockSpec((blk_rows, cols), lambda i: (i, 0)),
            pl.BlockSpec((blk_rows, cols), lambda i: (i, 0)),
        ],
        out_specs=pl.BlockSpec((1, 160, 128), lambda i: (i, 0, 0)),
        out_shape=jax.ShapeDtypeStruct((grid, 160, 128), jnp.float32),
        compiler_params=pltpu.CompilerParams(
            dimension_semantics=("parallel",)),
    )(xr, tr)

    sums = jnp.sum(out.reshape(grid, 20, 8 * 128), axis=(0, 2))
    t_j = sums[0:_BINS]
    c_i = sums[_BINS:2 * _BINS]
    nf = jnp.float32(n)
    tot = jnp.concatenate([jnp.array([nf], jnp.float32), c_i[:-1]]) - c_i
    loss_sum = t_j - jnp.concatenate([t_j[1:], jnp.zeros((1,), jnp.float32)])
    w = jnp.clip(tot, 1.0, None) ** jnp.float32(-_ALPHA)
    return jnp.sum(loss_sum * w) / nf


# speedup vs baseline: 1.7120x; 1.7120x over previous
import functools

import jax
import jax.numpy as jnp
import numpy as np
from jax.experimental import pallas as pl
from jax.experimental.pallas import tpu as pltpu

_BINS = 10
_ALPHA = 0.75
_EDGES = [np.float32(i) / np.float32(10.0) for i in range(_BINS + 1)]


def _ghm_body(x_ref, t_ref, out_ref, *, blk_rows):
    nchunks = blk_rows // 8

    def chunk(r0, accs):
        x = x_ref[pl.ds(r0, 8), :]
        t = t_ref[pl.ds(r0, 8), :]
        ax = jnp.abs(x)
        en = jnp.exp(-ax)
        loss = jnp.maximum(x, 0.0) - x * t + jnp.log1p(en)
        p1 = 1.0 / (1.0 + en)
        pred = jnp.where(x >= 0.0, p1, en * p1)
        g = jnp.abs(pred - t)

        new = list(accs)
        new[0] = new[0] + loss
        for i in range(1, _BINS + 1):
            m = g >= _EDGES[i]
            if i < _BINS:
                new[i] = new[i] + jnp.where(m, loss, 0.0)
            new[9 + i] = new[9 + i] + jnp.where(m, 1.0, 0.0)
        return tuple(new)

    zero = jnp.zeros((8, 128), jnp.float32)
    accs = (zero,) * 20
    for c in range(nchunks):
        accs = chunk(8 * c, accs)
    for j in range(20):
        out_ref[0, 8 * j:8 * j + 8, :] = accs[j]


def kernel(x, target):
    n = x.size
    cols = 128
    rows = n // cols
    blk_rows = min(1024, rows)
    grid = rows // blk_rows

    xr = x.reshape(rows, cols)
    tr = target.reshape(rows, cols)

    out = pl.pallas_call(
        functools.partial(_ghm_body, blk_rows=blk_rows),
        grid=(grid,),
        in_specs=[
            pl.BlockSpec((blk_rows, cols), lambda i: (i, 0)),
            pl.BlockSpec((blk_rows, cols), lambda i: (i, 0)),
        ],
        out_specs=pl.BlockSpec((1, 160, 128), lambda i: (i, 0, 0)),
        out_shape=jax.ShapeDtypeStruct((grid, 160, 128), jnp.float32),
        compiler_params=pltpu.CompilerParams(
            dimension_semantics=("parallel",)),
    )(xr, tr)

    sums = jnp.sum(out.reshape(grid, 20, 8 * 128), axis=(0, 2))
    t_j = sums[0:_BINS]
    c_i = sums[_BINS:2 * _BINS]
    nf = jnp.float32(n)
    tot = jnp.concatenate([jnp.array([nf], jnp.float32), c_i[:-1]]) - c_i
    loss_sum = t_j - jnp.concatenate([t_j[1:], jnp.zeros((1,), jnp.float32)])
    w = jnp.clip(tot, 1.0, None) ** jnp.float32(-_ALPHA)
    return jnp.sum(loss_sum * w) / nf
